# baseline (speedup 1.0000x reference)
import functools

import jax
import jax.numpy as jnp
from jax import lax
from jax.experimental import pallas as pl
from jax.experimental.pallas import tpu as pltpu
from jax.experimental.pallas import tpu_sc as plsc

N_IN_ROWS = 50000
N_OUT_ROWS = 400000
FEAT = 128

NUM_CORES = 2
NUM_SUBCORES = 16
NUM_WORKERS = NUM_CORES * NUM_SUBCORES

CHUNK = 128
NUM_CHUNKS = N_OUT_ROWS // CHUNK
STEPS = 98
NBUF = 7
DEPTH = 3


def _gather_body(table_hbm, idx_hbm, out_hbm, idxbufs, rowbufs, isems, gsems, wsems):
    wid = lax.axis_index("s") * NUM_CORES + lax.axis_index("c")

    def chunk_row0(k):
        c = wid * STEPS + k
        c = jnp.where(c < NUM_CHUNKS, c, c - NUM_CHUNKS)
        return c * CHUNK

    def idx_copy(k, b):
        return pltpu.make_async_copy(
            idx_hbm.at[pl.ds(chunk_row0(k), CHUNK)], idxbufs[b], isems[b]
        )

    def gather_copy(k, b):
        return pltpu.make_async_copy(
            table_hbm.at[idxbufs[b]], rowbufs[b], gsems[b]
        )

    def wb_copy(k, b):
        return pltpu.make_async_copy(
            rowbufs[b], out_hbm.at[pl.ds(chunk_row0(k), CHUNK)], wsems[b]
        )

    def step(k_static_b, g):
        b = k_static_b
        k = g * NBUF + b

        @pl.when(g >= 1)
        def _wait_buf_free():
            wb_copy(0, b).wait()

        idx_copy(k, b).start()

        b1 = (b - 1) % NBUF

        def _launch_gather():
            idx_copy(0, b1).wait()
            gather_copy(0, b1).start()

        if b >= 1:
            _launch_gather()
        else:
            pl.when(g >= 1)(_launch_gather)

        b2 = (b - 1 - DEPTH) % NBUF
        p = k - 1 - DEPTH

        def _writeback():
            gather_copy(0, b2).wait()
            wb_copy(p, b2).start()

        if b >= 1 + DEPTH:
            _writeback()
        else:
            pl.when(g >= 1)(_writeback)

    def ring_pass(g, _):
        for b in range(NBUF):
            step(b, g)
        return 0

    lax.fori_loop(0, STEPS // NBUF, ring_pass, 0)

    bl = (STEPS - 1) % NBUF
    idx_copy(0, bl).wait()
    gather_copy(0, bl).start()
    for p in range(STEPS - 1 - DEPTH, STEPS):
        pb = p % NBUF
        gather_copy(0, pb).wait()
        wb_copy(p, pb).start()
    for b in range(NBUF):
        wb_copy(0, b).wait()


@jax.jit
def _unpool_gather(table, idx):
    mesh = plsc.VectorSubcoreMesh(core_axis_name="c", subcore_axis_name="s")
    run = functools.partial(
        pl.kernel,
        mesh=mesh,
        out_type=jax.ShapeDtypeStruct((N_OUT_ROWS, FEAT), jnp.float32),
        scratch_types=[
            [pltpu.VMEM((CHUNK,), jnp.int32) for _ in range(NBUF)],
            [pltpu.VMEM((CHUNK, FEAT), jnp.float32) for _ in range(NBUF)],
            [pltpu.SemaphoreType.DMA for _ in range(NBUF)],
            [pltpu.SemaphoreType.DMA for _ in range(NBUF)],
            [pltpu.SemaphoreType.DMA for _ in range(NBUF)],
        ],
    )(_gather_body)
    return run(table, idx)


def kernel(input_features, unpool_map):
    return _unpool_gather(input_features, unpool_map.astype(jnp.int32))


# speedup vs baseline: 1.0009x; 1.0009x over previous
import functools

import jax
import jax.numpy as jnp
from jax import lax
from jax.experimental import pallas as pl
from jax.experimental.pallas import tpu as pltpu
from jax.experimental.pallas import tpu_sc as plsc

N_IN_ROWS = 50000
N_OUT_ROWS = 400000
FEAT = 128

NUM_CORES = 2
NUM_SUBCORES = 16
NUM_WORKERS = NUM_CORES * NUM_SUBCORES

CHUNK = 128
NUM_CHUNKS = N_OUT_ROWS // CHUNK
STEPS = 98
NBUF = 7
DEPTH = 2


def _gather_body(table_hbm, idx_hbm, out_hbm, idxbufs, rowbufs, isems, gsems, wsems):
    wid = lax.axis_index("s") * NUM_CORES + lax.axis_index("c")

    def chunk_row0(k):
        c = wid * STEPS + k
        c = jnp.where(c < NUM_CHUNKS, c, c - NUM_CHUNKS)
        return c * CHUNK

    def idx_copy(k, b):
        return pltpu.make_async_copy(
            idx_hbm.at[pl.ds(chunk_row0(k), CHUNK)], idxbufs[b], isems[b]
        )

    def gather_copy(k, b):
        return pltpu.make_async_copy(
            table_hbm.at[idxbufs[b]], rowbufs[b], gsems[b]
        )

    def wb_copy(k, b):
        return pltpu.make_async_copy(
            rowbufs[b], out_hbm.at[pl.ds(chunk_row0(k), CHUNK)], wsems[b]
        )

    def step(k_static_b, g):
        b = k_static_b
        k = g * NBUF + b

        @pl.when(g >= 1)
        def _wait_buf_free():
            wb_copy(0, b).wait()

        idx_copy(k, b).start()

        b1 = (b - 1) % NBUF

        def _launch_gather():
            idx_copy(0, b1).wait()
            gather_copy(0, b1).start()

        if b >= 1:
            _launch_gather()
        else:
            pl.when(g >= 1)(_launch_gather)

        b2 = (b - 1 - DEPTH) % NBUF
        p = k - 1 - DEPTH

        def _writeback():
            gather_copy(0, b2).wait()
            wb_copy(p, b2).start()

        if b >= 1 + DEPTH:
            _writeback()
        else:
            pl.when(g >= 1)(_writeback)

    def ring_pass(g, _):
        for b in range(NBUF):
            step(b, g)
        return 0

    lax.fori_loop(0, STEPS // NBUF, ring_pass, 0)

    bl = (STEPS - 1) % NBUF
    idx_copy(0, bl).wait()
    gather_copy(0, bl).start()
    for p in range(STEPS - 1 - DEPTH, STEPS):
        pb = p % NBUF
        gather_copy(0, pb).wait()
        wb_copy(p, pb).start()
    for b in range(NBUF):
        wb_copy(0, b).wait()


@jax.jit
def _unpool_gather(table, idx):
    mesh = plsc.VectorSubcoreMesh(core_axis_name="c", subcore_axis_name="s")
    run = functools.partial(
        pl.kernel,
        mesh=mesh,
        out_type=jax.ShapeDtypeStruct((N_OUT_ROWS, FEAT), jnp.float32),
        scratch_types=[
            [pltpu.VMEM((CHUNK,), jnp.int32) for _ in range(NBUF)],
            [pltpu.VMEM((CHUNK, FEAT), jnp.float32) for _ in range(NBUF)],
            [pltpu.SemaphoreType.DMA for _ in range(NBUF)],
            [pltpu.SemaphoreType.DMA for _ in range(NBUF)],
            [pltpu.SemaphoreType.DMA for _ in range(NBUF)],
        ],
    )(_gather_body)
    return run(table, idx)


def kernel(input_features, unpool_map):
    return _unpool_gather(input_features, unpool_map.astype(jnp.int32))
